# baseline (speedup 1.0000x reference)
import functools

import numpy as np
import jax
import jax.numpy as jnp
from jax import lax
from jax.experimental import pallas as pl
from jax.experimental.pallas import tpu as pltpu
from jax.experimental.pallas import tpu_sc as plsc

_G = 2
_R = 2
_NCH = _G * _R
_DIM = 1024
_DPG = _DIM // _G
_CDIM = 4
_NB = 8
_T = 2048
_TT = 512
_NT = _T // _TT
_TOKENS = _NB * _T
_EPS = np.float32(1e-5)

_HALF_L = np.float32(
    np.float32(np.float32(5.0) - np.float32(1.0))
    * np.float32(1.0 + 1e-3)
    / np.float32(2.0)
)

_NBINS = 625
_HB = 640
_NW = 16


def _gfsq_tc(x_ref, win_ref, bin_ref, wout_ref, bout_ref, feat_ref, ind_ref):
    xb = x_ref[0]
    ind_rows = []
    for g in range(_G):
        xg = xb[g * _DPG:(g + 1) * _DPG, :]
        wg = win_ref[g * _CDIM:(g + 1) * _CDIM, :]
        z = jax.lax.dot_general(
            wg, xg, (((1,), (0,)), ((), ())),
            preferred_element_type=jnp.float32)
        z = z + bin_ref[g * _CDIM:(g + 1) * _CDIM, 0:1]
        qout = jnp.zeros_like(z)
        r = z
        for i in range(_R):
            inv_scale = np.float32(4.0 ** i)
            scale = np.float32(4.0 ** (-i))
            q = jnp.round(jnp.tanh(r * inv_scale) * _HALF_L)
            codes = q * np.float32(0.5)
            zhat = q + np.float32(2.0)
            lo = zhat[0:1, :] + np.float32(5.0) * zhat[1:2, :]
            hi = zhat[2:3, :] + np.float32(5.0) * zhat[3:4, :]
            quant = codes * scale
            r = r - quant
            qout = qout + quant
            ind_rows.append(lo + np.float32(25.0) * hi)
        wo = wout_ref[g * _DPG:(g + 1) * _DPG, :]
        f = jax.lax.dot_general(
            wo, qout, (((1,), (0,)), ((), ())),
            preferred_element_type=jnp.float32)
        f = f + bout_ref[g * _DPG:(g + 1) * _DPG, 0:1]
        feat_ref[0, g * _DPG:(g + 1) * _DPG, :] = f
    ind_ref[0] = jnp.concatenate(
        [row.astype(jnp.int32) for row in ind_rows], axis=0)


def _ln16(x):
    bits = lax.bitcast_convert_type(x, jnp.int32)
    ex = lax.shift_right_arithmetic(bits, 23) - 127
    m = lax.bitcast_convert_type(
        (bits & jnp.int32(0x007FFFFF)) | jnp.int32(0x3F800000), jnp.float32)
    big = m > np.float32(1.4142135)
    m = jnp.where(big, m * np.float32(0.5), m)
    ex = ex + jnp.where(big, jnp.int32(1), jnp.int32(0))
    t = (m - np.float32(1.0)) / (m + np.float32(1.0))
    t2 = t * t
    p = t * (np.float32(2.0) + t2 * (
        np.float32(2.0 / 3.0) + t2 * (
            np.float32(2.0 / 5.0) + t2 * (
                np.float32(2.0 / 7.0) + t2 * np.float32(2.0 / 9.0)))))
    return p + ex.astype(jnp.float32) * np.float32(0.6931471805599453)


def _sc_stats(ind_hbm, perp_hbm, idx_v, hist_v, sum_v, res_v, big_v, shared_v):
    w = lax.axis_index("s")
    lane = lax.broadcasted_iota(jnp.int32, (16,), 0)
    base = lane * _HB
    ones = jnp.ones((16,), jnp.float32)
    zeros = jnp.zeros((16,), jnp.float32)
    c = w % _NCH
    b0 = w // _NCH

    def zero_body(j, carry):
        hist_v[pl.ds(j * 16, 16)] = zeros
        return carry
    lax.fori_loop(0, _NW * _HB // 16, zero_body, 0)

    for half in range(2):
        bb = b0 + half * (_NB // 2)
        pltpu.sync_copy(ind_hbm.at[bb, c], idx_v)

        def scat_body(j, carry):
            idx = idx_v[pl.ds(j * 16, 16)]
            plsc.addupdate_scatter(hist_v, [idx + base], ones)
            return carry
        lax.fori_loop(0, _T // 16, scat_body, 0)

    def fold_body(j, carry):
        acc = zeros
        for rl in range(_NW):
            acc = acc + hist_v[pl.ds(rl * _HB + j * 16, 16)]
        sum_v[pl.ds(j * 16, 16)] = acc
        return carry
    lax.fori_loop(0, _HB // 16, fold_body, 0)

    pltpu.sync_copy(sum_v, shared_v.at[w])
    plsc.subcore_barrier()

    @pl.when(w == 0)
    def _stats():
        pltpu.sync_copy(shared_v, big_v)
        res = jnp.zeros((16,), jnp.float32)
        inv_n = np.float32(1.0 / _TOKENS)
        for ch in range(_NCH):
            s_acc = zeros
            for j in range(_HB // 16):
                v = zeros
                for rw in range(ch, _NW, _NCH):
                    v = v + big_v[rw, pl.ds(j * 16, 16)]
                big_v[ch, pl.ds(j * 16, 16)] = v
                s_acc = s_acc + v
            s = jnp.sum(s_acc) * inv_n
            denom = s + _EPS
            ent_acc = zeros
            for j in range(_HB // 16):
                v = big_v[ch, pl.ds(j * 16, 16)]
                e = (v * inv_n) / denom
                ent_acc = ent_acc + e * _ln16(e + _EPS)
            hc = jnp.sum(ent_acc)
            res = jnp.where(lane == ch, -hc, res)
        res_v[...] = jnp.exp(res)
        pltpu.sync_copy(res_v, perp_hbm)


def _sc_call(ind):
    mesh = plsc.VectorSubcoreMesh(
        core_axis_name="c", subcore_axis_name="s", num_cores=1)
    f = functools.partial(
        pl.kernel,
        mesh=mesh,
        compiler_params=pltpu.CompilerParams(needs_layout_passes=False),
        out_type=jax.ShapeDtypeStruct((16,), jnp.float32),
        scratch_types=[
            pltpu.VMEM((_T,), jnp.int32),
            pltpu.VMEM((_NW * _HB,), jnp.float32),
            pltpu.VMEM((_HB,), jnp.float32),
            pltpu.VMEM((16,), jnp.float32),
            pltpu.VMEM((_NW, _HB), jnp.float32),
            pltpu.VMEM_SHARED((_NW, _HB), jnp.float32),
        ],
    )(_sc_stats)
    return f(ind)


def kernel(x, Win, bin_, Wout, bout):
    winr = Win.reshape(_G * _CDIM, _DPG)
    binr = bin_.reshape(_G * _CDIM, 1)
    woutr = Wout.reshape(_G * _DPG, _CDIM)
    boutr = bout.reshape(_G * _DPG, 1)
    feat, ind = pl.pallas_call(
        _gfsq_tc,
        grid=(_NB, _NT),
        in_specs=[
            pl.BlockSpec((1, _DIM, _TT), lambda b, t: (b, 0, t)),
            pl.BlockSpec((_G * _CDIM, _DPG), lambda b, t: (0, 0)),
            pl.BlockSpec((_G * _CDIM, 1), lambda b, t: (0, 0)),
            pl.BlockSpec((_G * _DPG, _CDIM), lambda b, t: (0, 0)),
            pl.BlockSpec((_G * _DPG, 1), lambda b, t: (0, 0)),
        ],
        out_specs=[
            pl.BlockSpec((1, _DIM, _TT), lambda b, t: (b, 0, t)),
            pl.BlockSpec((1, _NCH, _TT), lambda b, t: (b, 0, t)),
        ],
        out_shape=[
            jax.ShapeDtypeStruct((_NB, _DIM, _T), jnp.float32),
            jax.ShapeDtypeStruct((_NB, _NCH, _T), jnp.int32),
        ],
    )(x, winr, binr, woutr, boutr)
    perp16 = _sc_call(ind)
    p = perp16[:_NCH]
    return (jnp.zeros_like(p), feat, p, ind)


# speedup vs baseline: 1.1683x; 1.1683x over previous
import functools

import numpy as np
import jax
import jax.numpy as jnp
from jax import lax
from jax.experimental import pallas as pl
from jax.experimental.pallas import tpu as pltpu
from jax.experimental.pallas import tpu_sc as plsc

_G = 2
_R = 2
_NCH = _G * _R
_DIM = 1024
_DPG = _DIM // _G
_CDIM = 4
_NB = 8
_T = 2048
_TT = 2048
_NT = _T // _TT
_TOKENS = _NB * _T
_EPS = np.float32(1e-5)

_HALF_L = np.float32(
    np.float32(np.float32(5.0) - np.float32(1.0))
    * np.float32(1.0 + 1e-3)
    / np.float32(2.0)
)

_NBINS = 625
_HB = 640
_NW = 16


def _gfsq_tc(x_ref, win_ref, bin_ref, wout_ref, bout_ref, feat_ref, ind_ref):
    xb = x_ref[0]
    ind_rows = []
    for g in range(_G):
        xg = xb[g * _DPG:(g + 1) * _DPG, :]
        wg = win_ref[g * _CDIM:(g + 1) * _CDIM, :]
        z = jax.lax.dot_general(
            wg, xg, (((1,), (0,)), ((), ())),
            preferred_element_type=jnp.float32)
        z = z + bin_ref[g * _CDIM:(g + 1) * _CDIM, 0:1]
        qout = jnp.zeros_like(z)
        r = z
        for i in range(_R):
            inv_scale = np.float32(4.0 ** i)
            scale = np.float32(4.0 ** (-i))
            q = jnp.round(jnp.tanh(r * inv_scale) * _HALF_L)
            codes = q * np.float32(0.5)
            zhat = q + np.float32(2.0)
            lo = zhat[0:1, :] + np.float32(5.0) * zhat[1:2, :]
            hi = zhat[2:3, :] + np.float32(5.0) * zhat[3:4, :]
            quant = codes * scale
            r = r - quant
            qout = qout + quant
            ind_rows.append(lo + np.float32(25.0) * hi)
        wo = wout_ref[g * _DPG:(g + 1) * _DPG, :]
        f = jax.lax.dot_general(
            wo, qout, (((1,), (0,)), ((), ())),
            preferred_element_type=jnp.float32)
        f = f + bout_ref[g * _DPG:(g + 1) * _DPG, 0:1]
        feat_ref[0, g * _DPG:(g + 1) * _DPG, :] = f
    ind_ref[0] = jnp.concatenate(
        [row.astype(jnp.int32) for row in ind_rows], axis=0)


def _ln16(x):
    bits = lax.bitcast_convert_type(x, jnp.int32)
    ex = lax.shift_right_arithmetic(bits, 23) - 127
    m = lax.bitcast_convert_type(
        (bits & jnp.int32(0x007FFFFF)) | jnp.int32(0x3F800000), jnp.float32)
    big = m > np.float32(1.4142135)
    m = jnp.where(big, m * np.float32(0.5), m)
    ex = ex + jnp.where(big, jnp.int32(1), jnp.int32(0))
    t = (m - np.float32(1.0)) / (m + np.float32(1.0))
    t2 = t * t
    p = t * (np.float32(2.0) + t2 * (
        np.float32(2.0 / 3.0) + t2 * (
            np.float32(2.0 / 5.0) + t2 * (
                np.float32(2.0 / 7.0) + t2 * np.float32(2.0 / 9.0)))))
    return p + ex.astype(jnp.float32) * np.float32(0.6931471805599453)


def _sc_stats(ind_hbm, perp_hbm, idx_v, hist_v, sum_v, res_v, big_v, shared_v):
    w = lax.axis_index("s")
    lane = lax.broadcasted_iota(jnp.int32, (16,), 0)
    base = lane * _HB
    ones = jnp.ones((16,), jnp.float32)
    zeros = jnp.zeros((16,), jnp.float32)
    c = w % _NCH
    b0 = w // _NCH

    def zero_body(j, carry):
        hist_v[pl.ds(j * 16, 16)] = zeros
        return carry
    lax.fori_loop(0, _NW * _HB // 16, zero_body, 0)

    for half in range(2):
        bb = b0 + half * (_NB // 2)
        pltpu.sync_copy(ind_hbm.at[bb, c], idx_v)

        def scat_body(j, carry):
            idx = idx_v[pl.ds(j * 16, 16)]
            plsc.addupdate_scatter(hist_v, [idx + base], ones)
            return carry
        lax.fori_loop(0, _T // 16, scat_body, 0)

    def fold_body(j, carry):
        acc = zeros
        for rl in range(_NW):
            acc = acc + hist_v[pl.ds(rl * _HB + j * 16, 16)]
        sum_v[pl.ds(j * 16, 16)] = acc
        return carry
    lax.fori_loop(0, _HB // 16, fold_body, 0)

    pltpu.sync_copy(sum_v, shared_v.at[w])
    plsc.subcore_barrier()

    @pl.when(w == 0)
    def _stats():
        pltpu.sync_copy(shared_v, big_v)
        res = jnp.zeros((16,), jnp.float32)
        inv_n = np.float32(1.0 / _TOKENS)
        for ch in range(_NCH):
            s_acc = zeros
            for j in range(_HB // 16):
                v = zeros
                for rw in range(ch, _NW, _NCH):
                    v = v + big_v[rw, pl.ds(j * 16, 16)]
                big_v[ch, pl.ds(j * 16, 16)] = v
                s_acc = s_acc + v
            s = jnp.sum(s_acc) * inv_n
            denom = s + _EPS
            ent_acc = zeros
            for j in range(_HB // 16):
                v = big_v[ch, pl.ds(j * 16, 16)]
                e = (v * inv_n) / denom
                ent_acc = ent_acc + e * _ln16(e + _EPS)
            hc = jnp.sum(ent_acc)
            res = jnp.where(lane == ch, -hc, res)
        res_v[...] = jnp.exp(res)
        pltpu.sync_copy(res_v, perp_hbm)


def _sc_call(ind):
    mesh = plsc.VectorSubcoreMesh(
        core_axis_name="c", subcore_axis_name="s", num_cores=1)
    f = functools.partial(
        pl.kernel,
        mesh=mesh,
        compiler_params=pltpu.CompilerParams(needs_layout_passes=False),
        out_type=jax.ShapeDtypeStruct((16,), jnp.float32),
        scratch_types=[
            pltpu.VMEM((_T,), jnp.int32),
            pltpu.VMEM((_NW * _HB,), jnp.float32),
            pltpu.VMEM((_HB,), jnp.float32),
            pltpu.VMEM((16,), jnp.float32),
            pltpu.VMEM((_NW, _HB), jnp.float32),
            pltpu.VMEM_SHARED((_NW, _HB), jnp.float32),
        ],
    )(_sc_stats)
    return f(ind)


def kernel(x, Win, bin_, Wout, bout):
    winr = Win.reshape(_G * _CDIM, _DPG)
    binr = bin_.reshape(_G * _CDIM, 1)
    woutr = Wout.reshape(_G * _DPG, _CDIM)
    boutr = bout.reshape(_G * _DPG, 1)
    feat, ind = pl.pallas_call(
        _gfsq_tc,
        grid=(_NB, _NT),
        in_specs=[
            pl.BlockSpec((1, _DIM, _TT), lambda b, t: (b, 0, t)),
            pl.BlockSpec((_G * _CDIM, _DPG), lambda b, t: (0, 0)),
            pl.BlockSpec((_G * _CDIM, 1), lambda b, t: (0, 0)),
            pl.BlockSpec((_G * _DPG, _CDIM), lambda b, t: (0, 0)),
            pl.BlockSpec((_G * _DPG, 1), lambda b, t: (0, 0)),
        ],
        out_specs=[
            pl.BlockSpec((1, _DIM, _TT), lambda b, t: (b, 0, t)),
            pl.BlockSpec((1, _NCH, _TT), lambda b, t: (b, 0, t)),
        ],
        out_shape=[
            jax.ShapeDtypeStruct((_NB, _DIM, _T), jnp.float32),
            jax.ShapeDtypeStruct((_NB, _NCH, _T), jnp.int32),
        ],
    )(x, winr, binr, woutr, boutr)
    perp16 = _sc_call(ind)
    p = perp16[:_NCH]
    return (jnp.zeros_like(p), feat, p, ind)
